# baseline (speedup 1.0000x reference)
import functools

import jax
import jax.numpy as jnp
from jax import lax
from jax.experimental import pallas as pl
from jax.experimental.pallas import tpu as pltpu
from jax.experimental.pallas import tpu_sc as plsc

N = 10000
E = 320000
F_IN = 128
H = 500
HP = 512
K1 = 5000
K2 = 2500
OUT = 121
NSC = 2
NTILE = 16
EPT = E // NTILE
CH = 2000

_F32_FLIP = 0x7FFFFFFF
_I32_MIN = -2147483648


NW = NSC * NTILE
G = 16
CM = 2000
SUB = 80


def _make_edgemask():
    mesh = plsc.VectorSubcoreMesh(core_axis_name="c", subcore_axis_name="s",
                                  num_cores=NSC, num_subcores=NTILE)
    ept = E // NW

    @functools.partial(
        pl.kernel,
        mesh=mesh,
        compiler_params=pltpu.CompilerParams(needs_layout_passes=False),
        out_type=jax.ShapeDtypeStruct((E,), jnp.int32),
        scratch_types=[
            pltpu.VMEM((CM,), jnp.int32),
            pltpu.VMEM((CM,), jnp.int32),
            pltpu.VMEM((CM,), jnp.float32),
            pltpu.VMEM((CM,), jnp.float32),
            pltpu.VMEM((CM,), jnp.int32),
            pltpu.SemaphoreType.DMA,
        ],
    )
    def em(mask, src, dst, out, srcb, dstb, ms, md, ob, sem):
        c = lax.axis_index("c")
        s = lax.axis_index("s")
        wid = c * NTILE + s
        ebase = wid * ept

        def chunk(ci, _):
            off = ebase + ci * CM
            pltpu.sync_copy(src.at[pl.ds(off, CM)], srcb)
            pltpu.sync_copy(dst.at[pl.ds(off, CM)], dstb)
            for k in range(CM // SUB):
                pltpu.async_copy(mask.at[srcb.at[pl.ds(k * SUB, SUB)]],
                                 ms.at[pl.ds(k * SUB, SUB)], sem)
                pltpu.async_copy(mask.at[dstb.at[pl.ds(k * SUB, SUB)]],
                                 md.at[pl.ds(k * SUB, SUB)], sem)
            for k in range(CM // SUB):
                pltpu.make_async_copy(mask.at[srcb.at[pl.ds(k * SUB, SUB)]],
                                      ms.at[pl.ds(k * SUB, SUB)], sem).wait()
                pltpu.make_async_copy(mask.at[dstb.at[pl.ds(k * SUB, SUB)]],
                                      md.at[pl.ds(k * SUB, SUB)], sem).wait()

            def lp(i, _):
                sl = pl.ds(i * 16, 16)
                valid = (ms[sl] > 0.0) & (md[sl] > 0.0)
                ob[sl] = jnp.where(valid, dstb[sl], -1)
                return 0

            lax.fori_loop(0, CM // 16, lp, 0)
            pltpu.sync_copy(ob, out.at[pl.ds(off, CM)])
            return 0

        lax.fori_loop(0, ept // CM, chunk, 0)

    return em


def _make_segsum(feat, rpt, npass):
    mesh = plsc.VectorSubcoreMesh(core_axis_name="c", subcore_axis_name="s",
                                  num_cores=NSC, num_subcores=NTILE)
    nrows_out = NW * rpt * npass
    nch = E // CH
    funr = 8 if feat <= 128 else 4
    fblk = feat // (16 * funr)

    @functools.partial(
        pl.kernel,
        mesh=mesh,
        compiler_params=pltpu.CompilerParams(needs_layout_passes=False),
        out_type=jax.ShapeDtypeStruct((nrows_out * feat,), jnp.float32),
        scratch_types=[
            pltpu.VMEM((2 * CH,), jnp.int32),
            pltpu.VMEM((2 * CH,), jnp.int32),
            pltpu.VMEM((2 * CH,), jnp.float32),
            pltpu.VMEM((2 * (CH + 2 * G),), jnp.int32),
            pltpu.VMEM((2 * (CH + 2 * G),), jnp.int32),
            pltpu.VMEM((2 * (CH + 2 * G),), jnp.float32),
            pltpu.VMEM((G,), jnp.int32),
            pltpu.VMEM((G,), jnp.int32),
            pltpu.VMEM((G, feat), jnp.float32),
            pltpu.VMEM((G, feat), jnp.float32),
            pltpu.VMEM((rpt * feat,), jnp.float32),
            pltpu.SemaphoreType.DMA,
            pltpu.SemaphoreType.DMA,
            pltpu.SemaphoreType.DMA,
            pltpu.SemaphoreType.DMA,
        ],
    )
    def seg(tab, src, dst, ew, zrows, out,
            srcb, dstb, ewb, srcf, dstf, ewf, gidxa, gidxb, rowsa, rowsb,
            acc, sema, semb, semga, semgb):
        c = lax.axis_index("c")
        s = lax.axis_index("s")
        wid = c * NTILE + s

        def start_stage(ci, hb, sem):
            off = ci * CH
            pltpu.async_copy(src.at[pl.ds(off, CH)],
                             srcb.at[pl.ds(hb, CH)], sem)
            pltpu.async_copy(dst.at[pl.ds(off, CH)],
                             dstb.at[pl.ds(hb, CH)], sem)
            pltpu.async_copy(ew.at[pl.ds(off, CH)],
                             ewb.at[pl.ds(hb, CH)], sem)

        def wait_stage(hb, sem):
            pltpu.make_async_copy(src.at[pl.ds(0, CH)],
                                  srcb.at[pl.ds(hb, CH)], sem).wait()
            pltpu.make_async_copy(dst.at[pl.ds(0, CH)],
                                  dstb.at[pl.ds(hb, CH)], sem).wait()
            pltpu.make_async_copy(ew.at[pl.ds(0, CH)],
                                  ewb.at[pl.ds(hb, CH)], sem).wait()

        def fire(off, gidx, rows, semg):
            gidx[...] = srcf[pl.ds(off, 16)]
            pltpu.async_copy(tab.at[gidx], rows, semg)

        def accum(off, rows):
            wv = ewf[pl.ds(off, 16)]
            dv = dstf[pl.ds(off, 16)]
            for r in range(16):
                w = wv[r]
                dl = dv[r]

                def fb(fi, _, w=w, dl=dl, r=r):
                    fo = dl * feat + fi * (16 * funr)
                    ro = fi * (16 * funr)
                    for k in range(funr):
                        acc[pl.ds(fo + k * 16, 16)] = (
                            acc[pl.ds(fo + k * 16, 16)]
                            + rows[r, pl.ds(ro + k * 16, 16)] * w)
                    return 0

                lax.fori_loop(0, fblk, fb, 0)

        fcap = CH + 2 * G

        def pass_body(p, _):
            base = (p * NW + wid) * rpt
            pltpu.sync_copy(zrows, acc)
            start_stage(0, 0, sema)
            start_stage(jnp.minimum(1, nch - 1), CH, semb)

            def chunk_body(ci, qstate, base=base):
                h = lax.rem(ci, 2)
                hb = h * CH
                fb = h * fcap

                def active(_):
                    @pl.when(h == 0)
                    def _():
                        wait_stage(0, sema)

                    @pl.when(h == 1)
                    def _():
                        wait_stage(CH, semb)

                    def filt(i, cnt):
                        dv = dstb[pl.ds(hb + i * 16, 16)]
                        dloc = dv - base
                        m = (dloc >= 0) & (dloc < rpt)

                        def dofilt(cnt):
                            mi = jnp.where(m, 1, 0)
                            pos = fb + cnt + plsc.cumsum(mi) - 1
                            plsc.store_scatter(srcf, [pos],
                                               srcb[pl.ds(hb + i * 16, 16)],
                                               mask=m)
                            plsc.store_scatter(dstf, [pos], dloc, mask=m)
                            plsc.store_scatter(ewf, [pos],
                                               ewb[pl.ds(hb + i * 16, 16)],
                                               mask=m)
                            return pos[15] + 1 - fb

                        return lax.cond(jnp.any(m), dofilt,
                                        lambda cnt: cnt, cnt)

                    cnt = lax.fori_loop(0, CH // 16, filt, jnp.int32(0))
                    for t in range(2):
                        sl = pl.ds(fb + cnt + t * 16, 16)
                        srcf[sl] = jnp.zeros((16,), jnp.int32)
                        dstf[sl] = jnp.zeros((16,), jnp.int32)
                        ewf[sl] = jnp.zeros((16,), jnp.float32)
                    return jnp.where(cnt > 0,
                                     (((cnt + G - 1) // G + 1) // 2) * 2, 0)

                nck = lax.cond(ci < nch, active, lambda _: jnp.int32(0), 0)
                npair = nck // 2
                trips = jnp.where(ci == nch, 1, npair)

                def pairs(jj, qs, fb=fb, npair=npair):
                    qoff, qv = qs
                    firev = jj < npair
                    noff = fb + jj * 2 * G

                    @pl.when(qv)
                    def _():
                        pltpu.make_async_copy(tab.at[gidxa], rowsa,
                                              semga).wait()
                        accum(qoff, rowsa)
                        pltpu.make_async_copy(tab.at[gidxb], rowsb,
                                              semgb).wait()
                        accum(qoff + G, rowsb)

                    @pl.when(firev)
                    def _():
                        fire(noff, gidxa, rowsa, semga)
                        fire(noff + G, gidxb, rowsb, semgb)

                    return (noff, firev)

                qstate = lax.fori_loop(0, jnp.maximum(trips, 1), pairs,
                                       qstate)

                @pl.when(ci + 2 < nch)
                def _():
                    @pl.when(h == 0)
                    def _():
                        start_stage(ci + 2, 0, sema)

                    @pl.when(h == 1)
                    def _():
                        start_stage(ci + 2, CH, semb)

                return qstate

            lax.fori_loop(0, nch + 1, chunk_body,
                          (jnp.int32(0), jnp.bool_(False)))
            pltpu.sync_copy(acc, out.at[pl.ds(base * feat, rpt * feat)])
            return 0

        lax.fori_loop(0, npass, pass_body, 0)

    return seg


_BM = 1000


def _mm_body(a_ref, b_ref, wa_ref, wb_ref, bias_ref, p_ref, h_ref, s_ref):
    h = (jnp.dot(a_ref[...], wa_ref[...], preferred_element_type=jnp.float32)
         + jnp.dot(b_ref[...], wb_ref[...], preferred_element_type=jnp.float32)
         + bias_ref[...])
    h = jnp.maximum(h, 0.0)
    h_ref[...] = h
    p = p_ref[...]
    nrm = jnp.sqrt(jnp.sum(p * p))
    s_ref[...] = jnp.tanh(
        jnp.dot(h, p, preferred_element_type=jnp.float32) / nrm)


def _mm_score(a, b, wa, wb, bias, p):
    kd = a.shape[1]
    grid = N // _BM
    return pl.pallas_call(
        _mm_body,
        grid=(grid,),
        in_specs=[
            pl.BlockSpec((_BM, kd), lambda i: (i, 0)),
            pl.BlockSpec((_BM, kd), lambda i: (i, 0)),
            pl.BlockSpec((kd, HP), lambda i: (0, 0)),
            pl.BlockSpec((kd, HP), lambda i: (0, 0)),
            pl.BlockSpec((1, HP), lambda i: (0, 0)),
            pl.BlockSpec((HP, 1), lambda i: (0, 0)),
        ],
        out_specs=[
            pl.BlockSpec((_BM, HP), lambda i: (i, 0)),
            pl.BlockSpec((_BM, 1), lambda i: (i, 0)),
        ],
        out_shape=[
            jax.ShapeDtypeStruct((N, HP), jnp.float32),
            jax.ShapeDtypeStruct((N, 1), jnp.float32),
        ],
    )(a, b, wa, wb, bias, p)


_TR, _TCL = 8, 1250


def _ikey(v):
    b = lax.bitcast_convert_type(v, jnp.int32)
    return jnp.where(b >= 0, b, b ^ _F32_FLIP)


def _bisect_kth(key, k):
    def step(i, t):
        cand = t + lax.shift_left(jnp.int32(1), jnp.int32(31) - i)
        cnt = jnp.sum(jnp.where(key >= cand, 1, 0))
        return jnp.where(cnt >= k, cand, t)
    return lax.fori_loop(0, 32, step, jnp.int32(_I32_MIN))


def _topk_body(k, s_ref, t_ref, m_ref, sm_ref):
    s = s_ref[...]
    tk = t_ref[...]
    key = _ikey(s)
    kk = jnp.int32(k)

    t0 = _bisect_kth(key, kk)
    gt = key > t0
    eq = key == t0
    extra = kk - jnp.sum(jnp.where(gt, 1, 0))

    key1 = jnp.where(eq, _ikey(tk), jnp.int32(_I32_MIN))
    t1 = _bisect_kth(key1, extra)
    gt1 = eq & (key1 > t1)
    eq1 = eq & (key1 == t1)
    extra1 = extra - jnp.sum(jnp.where(gt1, 1, 0))

    idx = (lax.broadcasted_iota(jnp.int32, (_TR, _TCL), 0) * _TCL
           + lax.broadcasted_iota(jnp.int32, (_TR, _TCL), 1))

    def jstep(_, lohi):
        lo, hi = lohi
        mid = (lo + hi) // 2
        cnt = jnp.sum(jnp.where(eq1 & (idx < mid), 1, 0))
        return (jnp.where(cnt >= extra1, lo, mid),
                jnp.where(cnt >= extra1, mid, hi))

    _, j = lax.fori_loop(0, 15, jstep, (jnp.int32(0), jnp.int32(N)))

    m = gt | gt1 | (eq1 & (idx < j))
    mf = m.astype(jnp.float32)
    m_ref[...] = mf
    sm_ref[...] = s * mf


def _topk(s, tiekey, k):
    return pl.pallas_call(
        functools.partial(_topk_body, k),
        out_shape=[
            jax.ShapeDtypeStruct((_TR, _TCL), jnp.float32),
            jax.ShapeDtypeStruct((_TR, _TCL), jnp.float32),
        ],
    )(s, tiekey)


def _readout_body(store_hm, h_ref, sm_ref, m_ref, *out_refs):
    if store_hm:
        hm_ref, gmax_ref, gsum_ref = out_refs
    else:
        gmax_ref, gsum_ref = out_refs
    i = pl.program_id(0)
    hm = h_ref[...] * sm_ref[...]
    if store_hm:
        hm_ref[...] = hm
    blkmax = jnp.max(jnp.where(m_ref[...] > 0, hm, -3e38), axis=0,
                     keepdims=True)
    blksum = jnp.sum(hm, axis=0, keepdims=True)

    @pl.when(i == 0)
    def _():
        gmax_ref[...] = blkmax
        gsum_ref[...] = blksum

    @pl.when(i > 0)
    def _():
        gmax_ref[...] = jnp.maximum(gmax_ref[...], blkmax)
        gsum_ref[...] = gsum_ref[...] + blksum


def _readout(h, sm, m, store_hm):
    grid = N // _BM
    out_specs = [pl.BlockSpec((1, HP), lambda i: (0, 0)),
                 pl.BlockSpec((1, HP), lambda i: (0, 0))]
    out_shape = [jax.ShapeDtypeStruct((1, HP), jnp.float32),
                 jax.ShapeDtypeStruct((1, HP), jnp.float32)]
    if store_hm:
        out_specs.insert(0, pl.BlockSpec((_BM, HP), lambda i: (i, 0)))
        out_shape.insert(0, jax.ShapeDtypeStruct((N, HP), jnp.float32))
    return pl.pallas_call(
        functools.partial(_readout_body, store_hm),
        grid=(grid,),
        in_specs=[
            pl.BlockSpec((_BM, HP), lambda i: (i, 0)),
            pl.BlockSpec((_BM, 1), lambda i: (i, 0)),
            pl.BlockSpec((_BM, 1), lambda i: (i, 0)),
        ],
        out_specs=out_specs,
        out_shape=out_shape,
    )(h, sm, m)


_L2P = 4096
_BL2 = 512


def _head_body(g1max_ref, g1sum_ref, g2max_ref, g2sum_ref,
               wl1_ref, bl1_ref, wl2_ref, bl2_ref, wl3_ref, bl3_ref,
               o_ref, g_scr, a1_scr):
    kstep = pl.program_id(0)

    @pl.when(kstep == 0)
    def _():
        gmax = g1max_ref[0:1, 0:H] + g2max_ref[0:1, 0:H]
        gmean = (g1sum_ref[0:1, 0:H] / K1) + (g2sum_ref[0:1, 0:H] / K2)
        g = jnp.concatenate([gmax, gmean], axis=1)
        g_scr[...] = g
        a1_scr[...] = jnp.maximum(
            jnp.dot(g, wl1_ref[...], preferred_element_type=jnp.float32)
            + bl1_ref[...], 0.0)
        o_ref[...] = jnp.zeros_like(o_ref)

    a2 = jnp.maximum(
        jnp.dot(a1_scr[...], wl2_ref[...], preferred_element_type=jnp.float32)
        + bl2_ref[...], 0.0)
    o_ref[...] = o_ref[...] + jnp.dot(a2, wl3_ref[...],
                                      preferred_element_type=jnp.float32)

    @pl.when(kstep == (_L2P // _BL2) - 1)
    def _():
        o_ref[...] = jax.nn.sigmoid(o_ref[...] + bl3_ref[...])


def _head(g1max, g1sum, g2max, g2sum, wl1t, bl1, wl2t, bl2, wl3t, bl3):
    grid = _L2P // _BL2
    return pl.pallas_call(
        _head_body,
        grid=(grid,),
        in_specs=[
            pl.BlockSpec((1, HP), lambda i: (0, 0)),
            pl.BlockSpec((1, HP), lambda i: (0, 0)),
            pl.BlockSpec((1, HP), lambda i: (0, 0)),
            pl.BlockSpec((1, HP), lambda i: (0, 0)),
            pl.BlockSpec((2 * H, 2000), lambda i: (0, 0)),
            pl.BlockSpec((1, 2000), lambda i: (0, 0)),
            pl.BlockSpec((2000, _BL2), lambda i: (0, i)),
            pl.BlockSpec((1, _BL2), lambda i: (0, i)),
            pl.BlockSpec((_BL2, OUT), lambda i: (i, 0)),
            pl.BlockSpec((1, OUT), lambda i: (0, 0)),
        ],
        out_specs=pl.BlockSpec((1, OUT), lambda i: (0, 0)),
        out_shape=jax.ShapeDtypeStruct((1, OUT), jnp.float32),
        scratch_shapes=[
            pltpu.VMEM((1, 2 * H), jnp.float32),
            pltpu.VMEM((1, 2000), jnp.float32),
        ],
    )(g1max, g1sum, g2max, g2sum, wl1t, bl1, wl2t, bl2, wl3t, bl3)


_RPT1, _NP1 = 320, 1
_RPT2, _NP2 = 160, 2


@functools.cache
def _segsum(feat, rpt, npass):
    return _make_segsum(feat, rpt, npass)


@functools.cache
def _edgemask():
    return _make_edgemask()


def _padw(w, rows, cols):
    return jnp.pad(w, ((0, rows - w.shape[0]), (0, cols - w.shape[1])))


def kernel(x, edge_index, edge_attr, W_rel1, b_rel1, W_root1, p1,
           W_rel2, b_rel2, W_root2, p2, W_l1, b_l1, W_l2, b_l2, W_l3, b_l3):
    src = edge_index[0]
    dst = edge_index[1]
    ew = edge_attr

    wr1t = _padw(W_rel1.T, F_IN, HP)
    wt1t = _padw(W_root1.T, F_IN, HP)
    b1p = _padw(b_rel1[None, :], 1, HP)
    p1p = _padw(p1[:, None], HP, 1)
    wr2t = _padw(W_rel2.T, HP, HP)
    wt2t = _padw(W_root2.T, HP, HP)
    b2p = _padw(b_rel2[None, :], 1, HP)
    p2p = _padw(p2[:, None], HP, 1)

    z1 = jnp.zeros((_RPT1 * F_IN,), jnp.float32)
    z2 = jnp.zeros((_RPT2 * HP,), jnp.float32)

    agg1 = _segsum(F_IN, _RPT1, _NP1)(x, src, dst, ew, z1)
    agg1 = agg1.reshape(-1, F_IN)[:N]
    h, s1 = _mm_score(agg1, x, wr1t, wt1t, b1p, p1p)

    s1r = s1.reshape(_TR, _TCL)
    m1r, sm1r = _topk(s1r, jnp.zeros((_TR, _TCL), jnp.float32), K1)
    m1 = m1r.reshape(N, 1)
    sm1 = sm1r.reshape(N, 1)

    hm, g1max, g1sum = _readout(h, sm1, m1, True)

    dst2 = _edgemask()(m1r.reshape(N), src, dst)
    agg2 = _segsum(HP, _RPT2, _NP2)(hm, src, dst2, ew, z2)
    agg2 = agg2.reshape(-1, HP)[:N]
    h2, s2 = _mm_score(agg2, hm, wr2t, wt2t, b2p, p2p)

    s2m = jnp.where(m1 > 0, s2, -2.0)
    tie = jnp.where(m1r > 0, s1r, -2.0)
    m2r, sm2r = _topk(s2m.reshape(_TR, _TCL), tie, K2)
    m2 = m2r.reshape(N, 1)
    sm2 = sm2r.reshape(N, 1)

    g2max, g2sum = _readout(h2, sm2, m2, False)

    wl1t = W_l1.T
    wl2t = _padw(W_l2.T, 2000, _L2P)
    bl2p = _padw(b_l2[None, :], 1, _L2P)
    wl3t = _padw(W_l3.T, _L2P, OUT)
    return _head(g1max, g1sum, g2max, g2sum,
                 wl1t, b_l1[None, :], wl2t, bl2p, wl3t, b_l3[None, :])


# speedup vs baseline: 2.2694x; 2.2694x over previous
import functools

import jax
import jax.numpy as jnp
from jax import lax
from jax.experimental import pallas as pl
from jax.experimental.pallas import tpu as pltpu
from jax.experimental.pallas import tpu_sc as plsc

N = 10000
E = 320000
F_IN = 128
H = 500
HP = 512
K1 = 5000
K2 = 2500
OUT = 121
NSC = 2
NTILE = 16
EPT = E // NTILE
CH = 2000

_F32_FLIP = 0x7FFFFFFF
_I32_MIN = -2147483648


NW = NSC * NTILE
G = 16
CM = 2000
SUB = 80


def _make_edgemask():
    mesh = plsc.VectorSubcoreMesh(core_axis_name="c", subcore_axis_name="s",
                                  num_cores=NSC, num_subcores=NTILE)
    ept = E // NW

    @functools.partial(
        pl.kernel,
        mesh=mesh,
        compiler_params=pltpu.CompilerParams(needs_layout_passes=False),
        out_type=jax.ShapeDtypeStruct((E,), jnp.int32),
        scratch_types=[
            pltpu.VMEM((CM,), jnp.int32),
            pltpu.VMEM((CM,), jnp.int32),
            pltpu.VMEM((CM,), jnp.float32),
            pltpu.VMEM((CM,), jnp.float32),
            pltpu.VMEM((CM,), jnp.int32),
            pltpu.SemaphoreType.DMA,
        ],
    )
    def em(mask, src, dst, out, srcb, dstb, ms, md, ob, sem):
        c = lax.axis_index("c")
        s = lax.axis_index("s")
        wid = c * NTILE + s
        ebase = wid * ept

        def chunk(ci, _):
            off = ebase + ci * CM
            pltpu.sync_copy(src.at[pl.ds(off, CM)], srcb)
            pltpu.sync_copy(dst.at[pl.ds(off, CM)], dstb)
            for k in range(CM // SUB):
                pltpu.async_copy(mask.at[srcb.at[pl.ds(k * SUB, SUB)]],
                                 ms.at[pl.ds(k * SUB, SUB)], sem)
                pltpu.async_copy(mask.at[dstb.at[pl.ds(k * SUB, SUB)]],
                                 md.at[pl.ds(k * SUB, SUB)], sem)
            for k in range(CM // SUB):
                pltpu.make_async_copy(mask.at[srcb.at[pl.ds(k * SUB, SUB)]],
                                      ms.at[pl.ds(k * SUB, SUB)], sem).wait()
                pltpu.make_async_copy(mask.at[dstb.at[pl.ds(k * SUB, SUB)]],
                                      md.at[pl.ds(k * SUB, SUB)], sem).wait()

            def lp(i, _):
                sl = pl.ds(i * 16, 16)
                valid = (ms[sl] > 0.0) & (md[sl] > 0.0)
                ob[sl] = jnp.where(valid, dstb[sl], -1)
                return 0

            lax.fori_loop(0, CM // 16, lp, 0)
            pltpu.sync_copy(ob, out.at[pl.ds(off, CM)])
            return 0

        lax.fori_loop(0, ept // CM, chunk, 0)

    return em


def _make_segsum(feat, rpt, npass, ch, g):
    mesh = plsc.VectorSubcoreMesh(core_axis_name="c", subcore_axis_name="s",
                                  num_cores=NSC, num_subcores=NTILE)
    nrows_out = NW * rpt * npass
    nch = E // ch
    funr = 8
    fblk = feat // (16 * funr)

    @functools.partial(
        pl.kernel,
        mesh=mesh,
        compiler_params=pltpu.CompilerParams(needs_layout_passes=False),
        out_type=jax.ShapeDtypeStruct((nrows_out * feat,), jnp.float32),
        scratch_types=[
            pltpu.VMEM((2 * ch,), jnp.int32),
            pltpu.VMEM((2 * ch,), jnp.int32),
            pltpu.VMEM((2 * ch,), jnp.float32),
            pltpu.VMEM((ch + g,), jnp.int32),
            pltpu.VMEM((ch + g,), jnp.int32),
            pltpu.VMEM((ch + g,), jnp.float32),
            pltpu.VMEM((g,), jnp.int32),
            pltpu.VMEM((g,), jnp.int32),
            pltpu.VMEM((g, feat), jnp.float32),
            pltpu.VMEM((g, feat), jnp.float32),
            pltpu.VMEM((rpt * feat,), jnp.float32),
            pltpu.SemaphoreType.DMA,
            pltpu.SemaphoreType.DMA,
            pltpu.SemaphoreType.DMA,
            pltpu.SemaphoreType.DMA,
        ],
    )
    def seg(tab, src, dst, ew, zrows, out,
            srcb, dstb, ewb, srcf, dstf, ewf, gidxa, gidxb, rowsa, rowsb,
            acc, sema, semb, semga, semgb):
        c = lax.axis_index("c")
        s = lax.axis_index("s")
        wid = c * NTILE + s

        def start_stage(ci, hb, sem):
            off = ci * ch
            pltpu.async_copy(src.at[pl.ds(off, ch)],
                             srcb.at[pl.ds(hb, ch)], sem)
            pltpu.async_copy(dst.at[pl.ds(off, ch)],
                             dstb.at[pl.ds(hb, ch)], sem)
            pltpu.async_copy(ew.at[pl.ds(off, ch)],
                             ewb.at[pl.ds(hb, ch)], sem)

        def wait_stage(hb, sem):
            pltpu.make_async_copy(src.at[pl.ds(0, ch)],
                                  srcb.at[pl.ds(hb, ch)], sem).wait()
            pltpu.make_async_copy(dst.at[pl.ds(0, ch)],
                                  dstb.at[pl.ds(hb, ch)], sem).wait()
            pltpu.make_async_copy(ew.at[pl.ds(0, ch)],
                                  ewb.at[pl.ds(hb, ch)], sem).wait()

        def fire(j, gidx, rows, semg):
            for t in range(g // 16):
                gidx[pl.ds(t * 16, 16)] = srcf[pl.ds(j * g + t * 16, 16)]
            pltpu.async_copy(tab.at[gidx], rows, semg)

        def accum(j, rows):
            for gg in range(g // 16):
                wv = ewf[pl.ds(j * g + gg * 16, 16)]
                dv = dstf[pl.ds(j * g + gg * 16, 16)]
                for r in range(16):
                    w = wv[r]
                    dl = dv[r]

                    def fb(fi, _, w=w, dl=dl, r=r, gg=gg):
                        fo = dl * feat + fi * (16 * funr)
                        ro = fi * (16 * funr)
                        for k in range(funr):
                            acc[pl.ds(fo + k * 16, 16)] = (
                                acc[pl.ds(fo + k * 16, 16)]
                                + rows[gg * 16 + r, pl.ds(ro + k * 16, 16)]
                                * w)
                        return 0

                    lax.fori_loop(0, fblk, fb, 0)

        def pass_body(p, _):
            base = (p * NW + wid) * rpt
            pltpu.sync_copy(zrows, acc)
            start_stage(0, 0, sema)
            start_stage(jnp.minimum(1, nch - 1), ch, semb)

            def chunk_body(ci, _, base=base):
                h = lax.rem(ci, 2)
                hb = h * ch

                @pl.when(h == 0)
                def _():
                    wait_stage(0, sema)

                @pl.when(h == 1)
                def _():
                    wait_stage(ch, semb)

                def filt(i, cnt):
                    dv = dstb[pl.ds(hb + i * 16, 16)]
                    dloc = dv - base
                    m = (dloc >= 0) & (dloc < rpt)

                    def dofilt(cnt):
                        mi = jnp.where(m, 1, 0)
                        pos = cnt + plsc.cumsum(mi) - 1
                        plsc.store_scatter(srcf, [pos],
                                           srcb[pl.ds(hb + i * 16, 16)],
                                           mask=m)
                        plsc.store_scatter(dstf, [pos], dloc, mask=m)
                        plsc.store_scatter(ewf, [pos],
                                           ewb[pl.ds(hb + i * 16, 16)],
                                           mask=m)
                        return pos[15] + 1

                    return lax.cond(jnp.any(m), dofilt, lambda cnt: cnt, cnt)

                cnt = lax.fori_loop(0, ch // 16, filt, jnp.int32(0))
                for t in range(g // 16):
                    sl = pl.ds(cnt + t * 16, 16)
                    srcf[sl] = jnp.zeros((16,), jnp.int32)
                    dstf[sl] = jnp.zeros((16,), jnp.int32)
                    ewf[sl] = jnp.zeros((16,), jnp.float32)
                nck = (cnt + g - 1) // g

                @pl.when(nck > 0)
                def _():
                    fire(0, gidxa, rowsa, semga)

                @pl.when(nck > 1)
                def _():
                    fire(1, gidxb, rowsb, semgb)

                def gpair(jj, _):
                    j0 = jj * 2

                    @pl.when(j0 < nck)
                    def _():
                        pltpu.make_async_copy(tab.at[gidxa], rowsa,
                                              semga).wait()
                        accum(j0, rowsa)

                        @pl.when(j0 + 2 < nck)
                        def _():
                            fire(j0 + 2, gidxa, rowsa, semga)

                    @pl.when(j0 + 1 < nck)
                    def _():
                        pltpu.make_async_copy(tab.at[gidxb], rowsb,
                                              semgb).wait()
                        accum(j0 + 1, rowsb)

                        @pl.when(j0 + 3 < nck)
                        def _():
                            fire(j0 + 3, gidxb, rowsb, semgb)

                    return 0

                lax.fori_loop(0, (nck + 1) // 2, gpair, 0)

                nxt = jnp.minimum(ci + 2, nch - 1)

                @pl.when(h == 0)
                def _():
                    start_stage(nxt, 0, sema)

                @pl.when(h == 1)
                def _():
                    start_stage(nxt, ch, semb)

                return 0

            lax.fori_loop(0, nch, chunk_body, 0)
            wait_stage(0, sema)
            wait_stage(ch, semb)
            pltpu.sync_copy(acc, out.at[pl.ds(base * feat, rpt * feat)])
            return 0

        lax.fori_loop(0, npass, pass_body, 0)

    return seg


_BM = 1000


def _mm_body(a_ref, b_ref, wa_ref, wb_ref, bias_ref, p_ref, h_ref, s_ref):
    h = (jnp.dot(a_ref[...], wa_ref[...], preferred_element_type=jnp.float32)
         + jnp.dot(b_ref[...], wb_ref[...], preferred_element_type=jnp.float32)
         + bias_ref[...])
    h = jnp.maximum(h, 0.0)
    h_ref[...] = h
    p = p_ref[...]
    nrm = jnp.sqrt(jnp.sum(p * p))
    s_ref[...] = jnp.tanh(
        jnp.dot(h, p, preferred_element_type=jnp.float32) / nrm)


def _mm_score(a, b, wa, wb, bias, p):
    kd = a.shape[1]
    grid = N // _BM
    return pl.pallas_call(
        _mm_body,
        grid=(grid,),
        in_specs=[
            pl.BlockSpec((_BM, kd), lambda i: (i, 0)),
            pl.BlockSpec((_BM, kd), lambda i: (i, 0)),
            pl.BlockSpec((kd, HP), lambda i: (0, 0)),
            pl.BlockSpec((kd, HP), lambda i: (0, 0)),
            pl.BlockSpec((1, HP), lambda i: (0, 0)),
            pl.BlockSpec((HP, 1), lambda i: (0, 0)),
        ],
        out_specs=[
            pl.BlockSpec((_BM, HP), lambda i: (i, 0)),
            pl.BlockSpec((_BM, 1), lambda i: (i, 0)),
        ],
        out_shape=[
            jax.ShapeDtypeStruct((N, HP), jnp.float32),
            jax.ShapeDtypeStruct((N, 1), jnp.float32),
        ],
    )(a, b, wa, wb, bias, p)


_TR, _TCL = 8, 1250


def _ikey(v):
    b = lax.bitcast_convert_type(v, jnp.int32)
    return jnp.where(b >= 0, b, b ^ _F32_FLIP)


def _bisect_kth(key, k):
    def step(i, t):
        cand = t + lax.shift_left(jnp.int32(1), jnp.int32(31) - i)
        cnt = jnp.sum(jnp.where(key >= cand, 1, 0))
        return jnp.where(cnt >= k, cand, t)
    return lax.fori_loop(0, 32, step, jnp.int32(_I32_MIN))


def _topk_body(k, s_ref, t_ref, m_ref, sm_ref):
    s = s_ref[...]
    tk = t_ref[...]
    key = _ikey(s)
    kk = jnp.int32(k)

    t0 = _bisect_kth(key, kk)
    gt = key > t0
    eq = key == t0
    extra = kk - jnp.sum(jnp.where(gt, 1, 0))

    key1 = jnp.where(eq, _ikey(tk), jnp.int32(_I32_MIN))
    t1 = _bisect_kth(key1, extra)
    gt1 = eq & (key1 > t1)
    eq1 = eq & (key1 == t1)
    extra1 = extra - jnp.sum(jnp.where(gt1, 1, 0))

    idx = (lax.broadcasted_iota(jnp.int32, (_TR, _TCL), 0) * _TCL
           + lax.broadcasted_iota(jnp.int32, (_TR, _TCL), 1))

    def jstep(_, lohi):
        lo, hi = lohi
        mid = (lo + hi) // 2
        cnt = jnp.sum(jnp.where(eq1 & (idx < mid), 1, 0))
        return (jnp.where(cnt >= extra1, lo, mid),
                jnp.where(cnt >= extra1, mid, hi))

    _, j = lax.fori_loop(0, 15, jstep, (jnp.int32(0), jnp.int32(N)))

    m = gt | gt1 | (eq1 & (idx < j))
    mf = m.astype(jnp.float32)
    m_ref[...] = mf
    sm_ref[...] = s * mf


def _topk(s, tiekey, k):
    return pl.pallas_call(
        functools.partial(_topk_body, k),
        out_shape=[
            jax.ShapeDtypeStruct((_TR, _TCL), jnp.float32),
            jax.ShapeDtypeStruct((_TR, _TCL), jnp.float32),
        ],
    )(s, tiekey)


def _readout_body(store_hm, h_ref, sm_ref, m_ref, *out_refs):
    if store_hm:
        hm_ref, gmax_ref, gsum_ref = out_refs
    else:
        gmax_ref, gsum_ref = out_refs
    i = pl.program_id(0)
    hm = h_ref[...] * sm_ref[...]
    if store_hm:
        hm_ref[...] = hm
    blkmax = jnp.max(jnp.where(m_ref[...] > 0, hm, -3e38), axis=0,
                     keepdims=True)
    blksum = jnp.sum(hm, axis=0, keepdims=True)

    @pl.when(i == 0)
    def _():
        gmax_ref[...] = blkmax
        gsum_ref[...] = blksum

    @pl.when(i > 0)
    def _():
        gmax_ref[...] = jnp.maximum(gmax_ref[...], blkmax)
        gsum_ref[...] = gsum_ref[...] + blksum


def _readout(h, sm, m, store_hm):
    grid = N // _BM
    out_specs = [pl.BlockSpec((1, HP), lambda i: (0, 0)),
                 pl.BlockSpec((1, HP), lambda i: (0, 0))]
    out_shape = [jax.ShapeDtypeStruct((1, HP), jnp.float32),
                 jax.ShapeDtypeStruct((1, HP), jnp.float32)]
    if store_hm:
        out_specs.insert(0, pl.BlockSpec((_BM, HP), lambda i: (i, 0)))
        out_shape.insert(0, jax.ShapeDtypeStruct((N, HP), jnp.float32))
    return pl.pallas_call(
        functools.partial(_readout_body, store_hm),
        grid=(grid,),
        in_specs=[
            pl.BlockSpec((_BM, HP), lambda i: (i, 0)),
            pl.BlockSpec((_BM, 1), lambda i: (i, 0)),
            pl.BlockSpec((_BM, 1), lambda i: (i, 0)),
        ],
        out_specs=out_specs,
        out_shape=out_shape,
    )(h, sm, m)


_L2P = 4096
_BL2 = 512


def _head_body(g1max_ref, g1sum_ref, g2max_ref, g2sum_ref,
               wl1_ref, bl1_ref, wl2_ref, bl2_ref, wl3_ref, bl3_ref,
               o_ref, g_scr, a1_scr):
    kstep = pl.program_id(0)

    @pl.when(kstep == 0)
    def _():
        gmax = g1max_ref[0:1, 0:H] + g2max_ref[0:1, 0:H]
        gmean = (g1sum_ref[0:1, 0:H] / K1) + (g2sum_ref[0:1, 0:H] / K2)
        g = jnp.concatenate([gmax, gmean], axis=1)
        g_scr[...] = g
        a1_scr[...] = jnp.maximum(
            jnp.dot(g, wl1_ref[...], preferred_element_type=jnp.float32)
            + bl1_ref[...], 0.0)
        o_ref[...] = jnp.zeros_like(o_ref)

    a2 = jnp.maximum(
        jnp.dot(a1_scr[...], wl2_ref[...], preferred_element_type=jnp.float32)
        + bl2_ref[...], 0.0)
    o_ref[...] = o_ref[...] + jnp.dot(a2, wl3_ref[...],
                                      preferred_element_type=jnp.float32)

    @pl.when(kstep == (_L2P // _BL2) - 1)
    def _():
        o_ref[...] = jax.nn.sigmoid(o_ref[...] + bl3_ref[...])


def _head(g1max, g1sum, g2max, g2sum, wl1t, bl1, wl2t, bl2, wl3t, bl3):
    grid = _L2P // _BL2
    return pl.pallas_call(
        _head_body,
        grid=(grid,),
        in_specs=[
            pl.BlockSpec((1, HP), lambda i: (0, 0)),
            pl.BlockSpec((1, HP), lambda i: (0, 0)),
            pl.BlockSpec((1, HP), lambda i: (0, 0)),
            pl.BlockSpec((1, HP), lambda i: (0, 0)),
            pl.BlockSpec((2 * H, 2000), lambda i: (0, 0)),
            pl.BlockSpec((1, 2000), lambda i: (0, 0)),
            pl.BlockSpec((2000, _BL2), lambda i: (0, i)),
            pl.BlockSpec((1, _BL2), lambda i: (0, i)),
            pl.BlockSpec((_BL2, OUT), lambda i: (i, 0)),
            pl.BlockSpec((1, OUT), lambda i: (0, 0)),
        ],
        out_specs=pl.BlockSpec((1, OUT), lambda i: (0, 0)),
        out_shape=jax.ShapeDtypeStruct((1, OUT), jnp.float32),
        scratch_shapes=[
            pltpu.VMEM((1, 2 * H), jnp.float32),
            pltpu.VMEM((1, 2000), jnp.float32),
        ],
    )(g1max, g1sum, g2max, g2sum, wl1t, bl1, wl2t, bl2, wl3t, bl3)


_RPT1, _NP1 = 320, 1
_RPT2, _NP2 = 160, 2


@functools.cache
def _segsum(feat, rpt, npass, ch, g):
    return _make_segsum(feat, rpt, npass, ch, g)


@functools.cache
def _edgemask():
    return _make_edgemask()


def _padw(w, rows, cols):
    return jnp.pad(w, ((0, rows - w.shape[0]), (0, cols - w.shape[1])))


def kernel(x, edge_index, edge_attr, W_rel1, b_rel1, W_root1, p1,
           W_rel2, b_rel2, W_root2, p2, W_l1, b_l1, W_l2, b_l2, W_l3, b_l3):
    src = edge_index[0]
    dst = edge_index[1]
    ew = edge_attr

    wr1t = _padw(W_rel1.T, F_IN, HP)
    wt1t = _padw(W_root1.T, F_IN, HP)
    b1p = _padw(b_rel1[None, :], 1, HP)
    p1p = _padw(p1[:, None], HP, 1)
    wr2t = _padw(W_rel2.T, HP, HP)
    wt2t = _padw(W_root2.T, HP, HP)
    b2p = _padw(b_rel2[None, :], 1, HP)
    p2p = _padw(p2[:, None], HP, 1)

    z1 = jnp.zeros((_RPT1 * F_IN,), jnp.float32)
    z2 = jnp.zeros((_RPT2 * HP,), jnp.float32)

    agg1 = _segsum(F_IN, _RPT1, _NP1, 4000, 32)(x, src, dst, ew, z1)
    agg1 = agg1.reshape(-1, F_IN)[:N]
    h, s1 = _mm_score(agg1, x, wr1t, wt1t, b1p, p1p)

    s1r = s1.reshape(_TR, _TCL)
    m1r, sm1r = _topk(s1r, jnp.zeros((_TR, _TCL), jnp.float32), K1)
    m1 = m1r.reshape(N, 1)
    sm1 = sm1r.reshape(N, 1)

    hm, g1max, g1sum = _readout(h, sm1, m1, True)

    dst2 = _edgemask()(m1r.reshape(N), src, dst)
    agg2 = _segsum(HP, _RPT2, _NP2, 2000, 16)(hm, src, dst2, ew, z2)
    agg2 = agg2.reshape(-1, HP)[:N]
    h2, s2 = _mm_score(agg2, hm, wr2t, wt2t, b2p, p2p)

    s2m = jnp.where(m1 > 0, s2, -2.0)
    tie = jnp.where(m1r > 0, s1r, -2.0)
    m2r, sm2r = _topk(s2m.reshape(_TR, _TCL), tie, K2)
    m2 = m2r.reshape(N, 1)
    sm2 = sm2r.reshape(N, 1)

    g2max, g2sum = _readout(h2, sm2, m2, False)

    wl1t = W_l1.T
    wl2t = _padw(W_l2.T, 2000, _L2P)
    bl2p = _padw(b_l2[None, :], 1, _L2P)
    wl3t = _padw(W_l3.T, _L2P, OUT)
    return _head(g1max, g1sum, g2max, g2sum,
                 wl1t, b_l1[None, :], wl2t, bl2p, wl3t, b_l3[None, :])


# speedup vs baseline: 2.2922x; 1.0101x over previous
import functools

import jax
import jax.numpy as jnp
from jax import lax
from jax.experimental import pallas as pl
from jax.experimental.pallas import tpu as pltpu
from jax.experimental.pallas import tpu_sc as plsc

N = 10000
E = 320000
F_IN = 128
H = 500
HP = 512
K1 = 5000
K2 = 2500
OUT = 121
NSC = 2
NTILE = 16
EPT = E // NTILE
CH = 2000

_F32_FLIP = 0x7FFFFFFF
_I32_MIN = -2147483648


NW = NSC * NTILE
G = 16
CM = 2000
SUB = 80


def _make_edgemask():
    mesh = plsc.VectorSubcoreMesh(core_axis_name="c", subcore_axis_name="s",
                                  num_cores=NSC, num_subcores=NTILE)
    ept = E // NW

    @functools.partial(
        pl.kernel,
        mesh=mesh,
        compiler_params=pltpu.CompilerParams(needs_layout_passes=False),
        out_type=jax.ShapeDtypeStruct((E,), jnp.int32),
        scratch_types=[
            pltpu.VMEM((CM,), jnp.int32),
            pltpu.VMEM((CM,), jnp.int32),
            pltpu.VMEM((CM,), jnp.float32),
            pltpu.VMEM((CM,), jnp.float32),
            pltpu.VMEM((CM,), jnp.int32),
            pltpu.SemaphoreType.DMA,
        ],
    )
    def em(mask, src, dst, out, srcb, dstb, ms, md, ob, sem):
        c = lax.axis_index("c")
        s = lax.axis_index("s")
        wid = c * NTILE + s
        ebase = wid * ept

        def chunk(ci, _):
            off = ebase + ci * CM
            pltpu.sync_copy(src.at[pl.ds(off, CM)], srcb)
            pltpu.sync_copy(dst.at[pl.ds(off, CM)], dstb)
            for k in range(CM // SUB):
                pltpu.async_copy(mask.at[srcb.at[pl.ds(k * SUB, SUB)]],
                                 ms.at[pl.ds(k * SUB, SUB)], sem)
                pltpu.async_copy(mask.at[dstb.at[pl.ds(k * SUB, SUB)]],
                                 md.at[pl.ds(k * SUB, SUB)], sem)
            for k in range(CM // SUB):
                pltpu.make_async_copy(mask.at[srcb.at[pl.ds(k * SUB, SUB)]],
                                      ms.at[pl.ds(k * SUB, SUB)], sem).wait()
                pltpu.make_async_copy(mask.at[dstb.at[pl.ds(k * SUB, SUB)]],
                                      md.at[pl.ds(k * SUB, SUB)], sem).wait()

            def lp(i, _):
                sl = pl.ds(i * 16, 16)
                valid = (ms[sl] > 0.0) & (md[sl] > 0.0)
                ob[sl] = jnp.where(valid, dstb[sl], -1)
                return 0

            lax.fori_loop(0, CM // 16, lp, 0)
            pltpu.sync_copy(ob, out.at[pl.ds(off, CM)])
            return 0

        lax.fori_loop(0, ept // CM, chunk, 0)

    return em


def _make_segsum(feat, rpt, npass, ch, g, ns):
    mesh = plsc.VectorSubcoreMesh(core_axis_name="c", subcore_axis_name="s",
                                  num_cores=NSC, num_subcores=NTILE)
    nrows_out = NW * rpt * npass
    nch = E // ch
    funr = 8
    fblk = feat // (16 * funr)
    scratch = [
        pltpu.VMEM((2 * ch,), jnp.int32),
        pltpu.VMEM((2 * ch,), jnp.int32),
        pltpu.VMEM((2 * ch,), jnp.float32),
        pltpu.VMEM((ch + g,), jnp.int32),
        pltpu.VMEM((ch + g,), jnp.int32),
        pltpu.VMEM((ch + g,), jnp.float32),
    ]
    scratch += [pltpu.VMEM((g,), jnp.int32) for _ in range(ns)]
    scratch += [pltpu.VMEM((g, feat), jnp.float32) for _ in range(ns)]
    scratch += [pltpu.VMEM((rpt * feat,), jnp.float32)]
    scratch += [pltpu.SemaphoreType.DMA] * (2 + ns)

    @functools.partial(
        pl.kernel,
        mesh=mesh,
        compiler_params=pltpu.CompilerParams(needs_layout_passes=False),
        out_type=jax.ShapeDtypeStruct((nrows_out * feat,), jnp.float32),
        scratch_types=scratch,
    )
    def seg(tab, src, dst, ew, zrows, out, *scr):
        srcb, dstb, ewb, srcf, dstf, ewf = scr[:6]
        gidxs = scr[6:6 + ns]
        rowss = scr[6 + ns:6 + 2 * ns]
        acc = scr[6 + 2 * ns]
        sema, semb = scr[7 + 2 * ns], scr[8 + 2 * ns]
        semgs = scr[9 + 2 * ns:9 + 2 * ns + ns]
        c = lax.axis_index("c")
        s = lax.axis_index("s")
        wid = c * NTILE + s

        def start_stage(ci, hb, sem):
            off = ci * ch
            pltpu.async_copy(src.at[pl.ds(off, ch)],
                             srcb.at[pl.ds(hb, ch)], sem)
            pltpu.async_copy(dst.at[pl.ds(off, ch)],
                             dstb.at[pl.ds(hb, ch)], sem)
            pltpu.async_copy(ew.at[pl.ds(off, ch)],
                             ewb.at[pl.ds(hb, ch)], sem)

        def wait_stage(hb, sem):
            pltpu.make_async_copy(src.at[pl.ds(0, ch)],
                                  srcb.at[pl.ds(hb, ch)], sem).wait()
            pltpu.make_async_copy(dst.at[pl.ds(0, ch)],
                                  dstb.at[pl.ds(hb, ch)], sem).wait()
            pltpu.make_async_copy(ew.at[pl.ds(0, ch)],
                                  ewb.at[pl.ds(hb, ch)], sem).wait()

        def fire(j, gidx, rows, semg):
            for t in range(g // 16):
                gidx[pl.ds(t * 16, 16)] = srcf[pl.ds(j * g + t * 16, 16)]
            pltpu.async_copy(tab.at[gidx], rows, semg)

        def accum(j, rows):
            for gg in range(g // 16):
                wv = ewf[pl.ds(j * g + gg * 16, 16)]
                dv = dstf[pl.ds(j * g + gg * 16, 16)]
                for r in range(16):
                    w = wv[r]
                    dl = dv[r]

                    def fb(fi, _, w=w, dl=dl, r=r, gg=gg):
                        fo = dl * feat + fi * (16 * funr)
                        ro = fi * (16 * funr)
                        for k in range(funr):
                            acc[pl.ds(fo + k * 16, 16)] = (
                                acc[pl.ds(fo + k * 16, 16)]
                                + rows[gg * 16 + r, pl.ds(ro + k * 16, 16)]
                                * w)
                        return 0

                    lax.fori_loop(0, fblk, fb, 0)

        def pass_body(p, _):
            base = (p * NW + wid) * rpt
            pltpu.sync_copy(zrows, acc)
            start_stage(0, 0, sema)
            start_stage(jnp.minimum(1, nch - 1), ch, semb)

            def chunk_body(ci, _, base=base):
                h = lax.rem(ci, 2)
                hb = h * ch

                @pl.when(h == 0)
                def _():
                    wait_stage(0, sema)

                @pl.when(h == 1)
                def _():
                    wait_stage(ch, semb)

                def filt(i, cnt):
                    dv = dstb[pl.ds(hb + i * 16, 16)]
                    dloc = dv - base
                    m = (dloc >= 0) & (dloc < rpt)

                    def dofilt(cnt):
                        mi = jnp.where(m, 1, 0)
                        pos = cnt + plsc.cumsum(mi) - 1
                        plsc.store_scatter(srcf, [pos],
                                           srcb[pl.ds(hb + i * 16, 16)],
                                           mask=m)
                        plsc.store_scatter(dstf, [pos], dloc, mask=m)
                        plsc.store_scatter(ewf, [pos],
                                           ewb[pl.ds(hb + i * 16, 16)],
                                           mask=m)
                        return pos[15] + 1

                    return lax.cond(jnp.any(m), dofilt, lambda cnt: cnt, cnt)

                cnt = lax.fori_loop(0, ch // 16, filt, jnp.int32(0))
                for t in range(g // 16):
                    sl = pl.ds(cnt + t * 16, 16)
                    srcf[sl] = jnp.zeros((16,), jnp.int32)
                    dstf[sl] = jnp.zeros((16,), jnp.int32)
                    ewf[sl] = jnp.zeros((16,), jnp.float32)
                nck = (cnt + g - 1) // g

                for k in range(ns):
                    @pl.when(nck > k)
                    def _(k=k):
                        fire(k, gidxs[k], rowss[k], semgs[k])

                def gloop(jj, _):
                    j0 = jj * ns
                    for sl in range(ns):
                        @pl.when(j0 + sl < nck)
                        def _(sl=sl, j0=j0):
                            pltpu.make_async_copy(tab.at[gidxs[sl]],
                                                  rowss[sl],
                                                  semgs[sl]).wait()
                            accum(j0 + sl, rowss[sl])

                            @pl.when(j0 + sl + ns < nck)
                            def _():
                                fire(j0 + sl + ns, gidxs[sl], rowss[sl],
                                     semgs[sl])
                    return 0

                lax.fori_loop(0, (nck + ns - 1) // ns, gloop, 0)

                nxt = jnp.minimum(ci + 2, nch - 1)

                @pl.when(h == 0)
                def _():
                    start_stage(nxt, 0, sema)

                @pl.when(h == 1)
                def _():
                    start_stage(nxt, ch, semb)

                return 0

            lax.fori_loop(0, nch, chunk_body, 0)
            wait_stage(0, sema)
            wait_stage(ch, semb)
            pltpu.sync_copy(acc, out.at[pl.ds(base * feat, rpt * feat)])
            return 0

        lax.fori_loop(0, npass, pass_body, 0)

    return seg


_BM = 1000


def _mm_body(a_ref, b_ref, wa_ref, wb_ref, bias_ref, p_ref, h_ref, s_ref):
    h = (jnp.dot(a_ref[...], wa_ref[...], preferred_element_type=jnp.float32)
         + jnp.dot(b_ref[...], wb_ref[...], preferred_element_type=jnp.float32)
         + bias_ref[...])
    h = jnp.maximum(h, 0.0)
    h_ref[...] = h
    p = p_ref[...]
    nrm = jnp.sqrt(jnp.sum(p * p))
    s_ref[...] = jnp.tanh(
        jnp.dot(h, p, preferred_element_type=jnp.float32) / nrm)


def _mm_score(a, b, wa, wb, bias, p):
    kd = a.shape[1]
    grid = N // _BM
    return pl.pallas_call(
        _mm_body,
        grid=(grid,),
        in_specs=[
            pl.BlockSpec((_BM, kd), lambda i: (i, 0)),
            pl.BlockSpec((_BM, kd), lambda i: (i, 0)),
            pl.BlockSpec((kd, HP), lambda i: (0, 0)),
            pl.BlockSpec((kd, HP), lambda i: (0, 0)),
            pl.BlockSpec((1, HP), lambda i: (0, 0)),
            pl.BlockSpec((HP, 1), lambda i: (0, 0)),
        ],
        out_specs=[
            pl.BlockSpec((_BM, HP), lambda i: (i, 0)),
            pl.BlockSpec((_BM, 1), lambda i: (i, 0)),
        ],
        out_shape=[
            jax.ShapeDtypeStruct((N, HP), jnp.float32),
            jax.ShapeDtypeStruct((N, 1), jnp.float32),
        ],
    )(a, b, wa, wb, bias, p)


_TR, _TCL = 8, 1250


def _ikey(v):
    b = lax.bitcast_convert_type(v, jnp.int32)
    return jnp.where(b >= 0, b, b ^ _F32_FLIP)


def _bisect_kth(key, k):
    def step(i, t):
        cand = t + lax.shift_left(jnp.int32(1), jnp.int32(31) - i)
        cnt = jnp.sum(jnp.where(key >= cand, 1, 0))
        return jnp.where(cnt >= k, cand, t)
    return lax.fori_loop(0, 32, step, jnp.int32(_I32_MIN))


def _topk_body(k, s_ref, t_ref, m_ref, sm_ref):
    s = s_ref[...]
    tk = t_ref[...]
    key = _ikey(s)
    kk = jnp.int32(k)

    t0 = _bisect_kth(key, kk)
    gt = key > t0
    eq = key == t0
    extra = kk - jnp.sum(jnp.where(gt, 1, 0))

    key1 = jnp.where(eq, _ikey(tk), jnp.int32(_I32_MIN))
    t1 = _bisect_kth(key1, extra)
    gt1 = eq & (key1 > t1)
    eq1 = eq & (key1 == t1)
    extra1 = extra - jnp.sum(jnp.where(gt1, 1, 0))

    idx = (lax.broadcasted_iota(jnp.int32, (_TR, _TCL), 0) * _TCL
           + lax.broadcasted_iota(jnp.int32, (_TR, _TCL), 1))

    def jstep(_, lohi):
        lo, hi = lohi
        mid = (lo + hi) // 2
        cnt = jnp.sum(jnp.where(eq1 & (idx < mid), 1, 0))
        return (jnp.where(cnt >= extra1, lo, mid),
                jnp.where(cnt >= extra1, mid, hi))

    _, j = lax.fori_loop(0, 15, jstep, (jnp.int32(0), jnp.int32(N)))

    m = gt | gt1 | (eq1 & (idx < j))
    mf = m.astype(jnp.float32)
    m_ref[...] = mf
    sm_ref[...] = s * mf


def _topk(s, tiekey, k):
    return pl.pallas_call(
        functools.partial(_topk_body, k),
        out_shape=[
            jax.ShapeDtypeStruct((_TR, _TCL), jnp.float32),
            jax.ShapeDtypeStruct((_TR, _TCL), jnp.float32),
        ],
    )(s, tiekey)


def _readout_body(store_hm, h_ref, sm_ref, m_ref, *out_refs):
    if store_hm:
        hm_ref, gmax_ref, gsum_ref = out_refs
    else:
        gmax_ref, gsum_ref = out_refs
    i = pl.program_id(0)
    hm = h_ref[...] * sm_ref[...]
    if store_hm:
        hm_ref[...] = hm
    blkmax = jnp.max(jnp.where(m_ref[...] > 0, hm, -3e38), axis=0,
                     keepdims=True)
    blksum = jnp.sum(hm, axis=0, keepdims=True)

    @pl.when(i == 0)
    def _():
        gmax_ref[...] = blkmax
        gsum_ref[...] = blksum

    @pl.when(i > 0)
    def _():
        gmax_ref[...] = jnp.maximum(gmax_ref[...], blkmax)
        gsum_ref[...] = gsum_ref[...] + blksum


def _readout(h, sm, m, store_hm):
    grid = N // _BM
    out_specs = [pl.BlockSpec((1, HP), lambda i: (0, 0)),
                 pl.BlockSpec((1, HP), lambda i: (0, 0))]
    out_shape = [jax.ShapeDtypeStruct((1, HP), jnp.float32),
                 jax.ShapeDtypeStruct((1, HP), jnp.float32)]
    if store_hm:
        out_specs.insert(0, pl.BlockSpec((_BM, HP), lambda i: (i, 0)))
        out_shape.insert(0, jax.ShapeDtypeStruct((N, HP), jnp.float32))
    return pl.pallas_call(
        functools.partial(_readout_body, store_hm),
        grid=(grid,),
        in_specs=[
            pl.BlockSpec((_BM, HP), lambda i: (i, 0)),
            pl.BlockSpec((_BM, 1), lambda i: (i, 0)),
            pl.BlockSpec((_BM, 1), lambda i: (i, 0)),
        ],
        out_specs=out_specs,
        out_shape=out_shape,
    )(h, sm, m)


_L2P = 4096
_BL2 = 512


def _head_body(g1max_ref, g1sum_ref, g2max_ref, g2sum_ref,
               wl1_ref, bl1_ref, wl2_ref, bl2_ref, wl3_ref, bl3_ref,
               o_ref, g_scr, a1_scr):
    kstep = pl.program_id(0)

    @pl.when(kstep == 0)
    def _():
        gmax = g1max_ref[0:1, 0:H] + g2max_ref[0:1, 0:H]
        gmean = (g1sum_ref[0:1, 0:H] / K1) + (g2sum_ref[0:1, 0:H] / K2)
        g = jnp.concatenate([gmax, gmean], axis=1)
        g_scr[...] = g
        a1_scr[...] = jnp.maximum(
            jnp.dot(g, wl1_ref[...], preferred_element_type=jnp.float32)
            + bl1_ref[...], 0.0)
        o_ref[...] = jnp.zeros_like(o_ref)

    a2 = jnp.maximum(
        jnp.dot(a1_scr[...], wl2_ref[...], preferred_element_type=jnp.float32)
        + bl2_ref[...], 0.0)
    o_ref[...] = o_ref[...] + jnp.dot(a2, wl3_ref[...],
                                      preferred_element_type=jnp.float32)

    @pl.when(kstep == (_L2P // _BL2) - 1)
    def _():
        o_ref[...] = jax.nn.sigmoid(o_ref[...] + bl3_ref[...])


def _head(g1max, g1sum, g2max, g2sum, wl1t, bl1, wl2t, bl2, wl3t, bl3):
    grid = _L2P // _BL2
    return pl.pallas_call(
        _head_body,
        grid=(grid,),
        in_specs=[
            pl.BlockSpec((1, HP), lambda i: (0, 0)),
            pl.BlockSpec((1, HP), lambda i: (0, 0)),
            pl.BlockSpec((1, HP), lambda i: (0, 0)),
            pl.BlockSpec((1, HP), lambda i: (0, 0)),
            pl.BlockSpec((2 * H, 2000), lambda i: (0, 0)),
            pl.BlockSpec((1, 2000), lambda i: (0, 0)),
            pl.BlockSpec((2000, _BL2), lambda i: (0, i)),
            pl.BlockSpec((1, _BL2), lambda i: (0, i)),
            pl.BlockSpec((_BL2, OUT), lambda i: (i, 0)),
            pl.BlockSpec((1, OUT), lambda i: (0, 0)),
        ],
        out_specs=pl.BlockSpec((1, OUT), lambda i: (0, 0)),
        out_shape=jax.ShapeDtypeStruct((1, OUT), jnp.float32),
        scratch_shapes=[
            pltpu.VMEM((1, 2 * H), jnp.float32),
            pltpu.VMEM((1, 2000), jnp.float32),
        ],
    )(g1max, g1sum, g2max, g2sum, wl1t, bl1, wl2t, bl2, wl3t, bl3)


_RPT1, _NP1 = 320, 1
_RPT2, _NP2 = 160, 2


@functools.cache
def _segsum(feat, rpt, npass, ch, g, ns):
    return _make_segsum(feat, rpt, npass, ch, g, ns)


@functools.cache
def _edgemask():
    return _make_edgemask()


def _padw(w, rows, cols):
    return jnp.pad(w, ((0, rows - w.shape[0]), (0, cols - w.shape[1])))


def kernel(x, edge_index, edge_attr, W_rel1, b_rel1, W_root1, p1,
           W_rel2, b_rel2, W_root2, p2, W_l1, b_l1, W_l2, b_l2, W_l3, b_l3):
    src = edge_index[0]
    dst = edge_index[1]
    ew = edge_attr

    wr1t = _padw(W_rel1.T, F_IN, HP)
    wt1t = _padw(W_root1.T, F_IN, HP)
    b1p = _padw(b_rel1[None, :], 1, HP)
    p1p = _padw(p1[:, None], HP, 1)
    wr2t = _padw(W_rel2.T, HP, HP)
    wt2t = _padw(W_root2.T, HP, HP)
    b2p = _padw(b_rel2[None, :], 1, HP)
    p2p = _padw(p2[:, None], HP, 1)

    z1 = jnp.zeros((_RPT1 * F_IN,), jnp.float32)
    z2 = jnp.zeros((_RPT2 * HP,), jnp.float32)

    agg1 = _segsum(F_IN, _RPT1, _NP1, 2000, 16, 4)(x, src, dst, ew, z1)
    agg1 = agg1.reshape(-1, F_IN)[:N]
    h, s1 = _mm_score(agg1, x, wr1t, wt1t, b1p, p1p)

    s1r = s1.reshape(_TR, _TCL)
    m1r, sm1r = _topk(s1r, jnp.zeros((_TR, _TCL), jnp.float32), K1)
    m1 = m1r.reshape(N, 1)
    sm1 = sm1r.reshape(N, 1)

    hm, g1max, g1sum = _readout(h, sm1, m1, True)

    dst2 = _edgemask()(m1r.reshape(N), src, dst)
    agg2 = _segsum(HP, _RPT2, _NP2, 2000, 16, 2)(hm, src, dst2, ew, z2)
    agg2 = agg2.reshape(-1, HP)[:N]
    h2, s2 = _mm_score(agg2, hm, wr2t, wt2t, b2p, p2p)

    s2m = jnp.where(m1 > 0, s2, -2.0)
    tie = jnp.where(m1r > 0, s1r, -2.0)
    m2r, sm2r = _topk(s2m.reshape(_TR, _TCL), tie, K2)
    m2 = m2r.reshape(N, 1)
    sm2 = sm2r.reshape(N, 1)

    g2max, g2sum = _readout(h2, sm2, m2, False)

    wl1t = W_l1.T
    wl2t = _padw(W_l2.T, 2000, _L2P)
    bl2p = _padw(b_l2[None, :], 1, _L2P)
    wl3t = _padw(W_l3.T, _L2P, OUT)
    return _head(g1max, g1sum, g2max, g2sum,
                 wl1t, b_l1[None, :], wl2t, bl2p, wl3t, b_l3[None, :])


# speedup vs baseline: 2.3126x; 1.0089x over previous
import functools

import jax
import jax.numpy as jnp
from jax import lax
from jax.experimental import pallas as pl
from jax.experimental.pallas import tpu as pltpu
from jax.experimental.pallas import tpu_sc as plsc

N = 10000
E = 320000
F_IN = 128
H = 500
HP = 512
K1 = 5000
K2 = 2500
OUT = 121
NSC = 2
NTILE = 16
EPT = E // NTILE
CH = 2000

_F32_FLIP = 0x7FFFFFFF
_I32_MIN = -2147483648


NW = NSC * NTILE
G = 16
CM = 2000
SUB = 80


def _make_edgemask():
    mesh = plsc.VectorSubcoreMesh(core_axis_name="c", subcore_axis_name="s",
                                  num_cores=NSC, num_subcores=NTILE)
    ept = E // NW

    @functools.partial(
        pl.kernel,
        mesh=mesh,
        compiler_params=pltpu.CompilerParams(needs_layout_passes=False),
        out_type=jax.ShapeDtypeStruct((E,), jnp.int32),
        scratch_types=[
            pltpu.VMEM((CM,), jnp.int32),
            pltpu.VMEM((CM,), jnp.int32),
            pltpu.VMEM((CM,), jnp.float32),
            pltpu.VMEM((CM,), jnp.float32),
            pltpu.VMEM((CM,), jnp.int32),
            pltpu.SemaphoreType.DMA,
        ],
    )
    def em(mask, src, dst, out, srcb, dstb, ms, md, ob, sem):
        c = lax.axis_index("c")
        s = lax.axis_index("s")
        wid = c * NTILE + s
        ebase = wid * ept

        def chunk(ci, _):
            off = ebase + ci * CM
            pltpu.sync_copy(src.at[pl.ds(off, CM)], srcb)
            pltpu.sync_copy(dst.at[pl.ds(off, CM)], dstb)
            for k in range(CM // SUB):
                pltpu.async_copy(mask.at[srcb.at[pl.ds(k * SUB, SUB)]],
                                 ms.at[pl.ds(k * SUB, SUB)], sem)
                pltpu.async_copy(mask.at[dstb.at[pl.ds(k * SUB, SUB)]],
                                 md.at[pl.ds(k * SUB, SUB)], sem)
            for k in range(CM // SUB):
                pltpu.make_async_copy(mask.at[srcb.at[pl.ds(k * SUB, SUB)]],
                                      ms.at[pl.ds(k * SUB, SUB)], sem).wait()
                pltpu.make_async_copy(mask.at[dstb.at[pl.ds(k * SUB, SUB)]],
                                      md.at[pl.ds(k * SUB, SUB)], sem).wait()

            def lp(i, _):
                sl = pl.ds(i * 16, 16)
                valid = (ms[sl] > 0.0) & (md[sl] > 0.0)
                ob[sl] = jnp.where(valid, dstb[sl], -1)
                return 0

            lax.fori_loop(0, CM // 16, lp, 0)
            pltpu.sync_copy(ob, out.at[pl.ds(off, CM)])
            return 0

        lax.fori_loop(0, ept // CM, chunk, 0)

    return em


def _make_segsum(feat, rpt, npass, ch, g, ns):
    mesh = plsc.VectorSubcoreMesh(core_axis_name="c", subcore_axis_name="s",
                                  num_cores=NSC, num_subcores=NTILE)
    nrows_out = NW * rpt * npass
    nch = E // ch
    funr = 8
    fblk = feat // (16 * funr)
    scratch = [
        pltpu.VMEM((2 * ch,), jnp.int32),
        pltpu.VMEM((2 * ch,), jnp.int32),
        pltpu.VMEM((2 * ch,), jnp.float32),
        pltpu.VMEM((ch + g,), jnp.int32),
        pltpu.VMEM((ch + g,), jnp.int32),
        pltpu.VMEM((ch + g,), jnp.float32),
    ]
    scratch += [pltpu.VMEM((g,), jnp.int32) for _ in range(ns)]
    scratch += [pltpu.VMEM((g, feat), jnp.float32) for _ in range(ns)]
    scratch += [pltpu.VMEM((rpt * feat,), jnp.float32)]
    scratch += [pltpu.SemaphoreType.DMA] * (2 + ns)

    @functools.partial(
        pl.kernel,
        mesh=mesh,
        compiler_params=pltpu.CompilerParams(needs_layout_passes=False),
        out_type=jax.ShapeDtypeStruct((nrows_out * feat,), jnp.float32),
        scratch_types=scratch,
    )
    def seg(tab, src, dst, ew, zrows, out, *scr):
        srcb, dstb, ewb, srcf, dstf, ewf = scr[:6]
        gidxs = scr[6:6 + ns]
        rowss = scr[6 + ns:6 + 2 * ns]
        acc = scr[6 + 2 * ns]
        sema, semb = scr[7 + 2 * ns], scr[8 + 2 * ns]
        semgs = scr[9 + 2 * ns:9 + 2 * ns + ns]
        c = lax.axis_index("c")
        s = lax.axis_index("s")
        wid = c * NTILE + s

        def start_stage(ci, hb, sem):
            off = ci * ch
            pltpu.async_copy(src.at[pl.ds(off, ch)],
                             srcb.at[pl.ds(hb, ch)], sem)
            pltpu.async_copy(dst.at[pl.ds(off, ch)],
                             dstb.at[pl.ds(hb, ch)], sem)
            pltpu.async_copy(ew.at[pl.ds(off, ch)],
                             ewb.at[pl.ds(hb, ch)], sem)

        def wait_stage(hb, sem):
            pltpu.make_async_copy(src.at[pl.ds(0, ch)],
                                  srcb.at[pl.ds(hb, ch)], sem).wait()
            pltpu.make_async_copy(dst.at[pl.ds(0, ch)],
                                  dstb.at[pl.ds(hb, ch)], sem).wait()
            pltpu.make_async_copy(ew.at[pl.ds(0, ch)],
                                  ewb.at[pl.ds(hb, ch)], sem).wait()

        def fire(j, gidx, rows, semg):
            for t in range(g // 16):
                gidx[pl.ds(t * 16, 16)] = srcf[pl.ds(j * g + t * 16, 16)]
            pltpu.async_copy(tab.at[gidx], rows, semg)

        def accum(j, rows):
            for gg in range(g // 16):
                wv = ewf[pl.ds(j * g + gg * 16, 16)]
                dv = dstf[pl.ds(j * g + gg * 16, 16)]
                for r in range(16):
                    w = wv[r]
                    dl = dv[r]

                    def fb(fi, _, w=w, dl=dl, r=r, gg=gg):
                        fo = dl * feat + fi * (16 * funr)
                        ro = fi * (16 * funr)
                        for k in range(funr):
                            acc[pl.ds(fo + k * 16, 16)] = (
                                acc[pl.ds(fo + k * 16, 16)]
                                + rows[gg * 16 + r, pl.ds(ro + k * 16, 16)]
                                * w)
                        return 0

                    lax.fori_loop(0, fblk, fb, 0)

        def pass_body(p, _):
            base = (p * NW + wid) * rpt
            pltpu.sync_copy(zrows, acc)
            start_stage(0, 0, sema)
            start_stage(jnp.minimum(1, nch - 1), ch, semb)

            def chunk_body(ci, _, base=base):
                h = lax.rem(ci, 2)
                hb = h * ch

                @pl.when(h == 0)
                def _():
                    wait_stage(0, sema)

                @pl.when(h == 1)
                def _():
                    wait_stage(ch, semb)

                def filt(i, cnt):
                    dv = dstb[pl.ds(hb + i * 16, 16)]
                    dloc = dv - base
                    m = (dloc >= 0) & (dloc < rpt)

                    def dofilt(cnt):
                        mi = jnp.where(m, 1, 0)
                        pos = cnt + plsc.cumsum(mi) - 1
                        plsc.store_scatter(srcf, [pos],
                                           srcb[pl.ds(hb + i * 16, 16)],
                                           mask=m)
                        plsc.store_scatter(dstf, [pos], dloc, mask=m)
                        plsc.store_scatter(ewf, [pos],
                                           ewb[pl.ds(hb + i * 16, 16)],
                                           mask=m)
                        return pos[15] + 1

                    return lax.cond(jnp.any(m), dofilt, lambda cnt: cnt, cnt)

                cnt = lax.fori_loop(0, ch // 16, filt, jnp.int32(0))
                for t in range(g // 16):
                    sl = pl.ds(cnt + t * 16, 16)
                    srcf[sl] = jnp.zeros((16,), jnp.int32)
                    dstf[sl] = jnp.zeros((16,), jnp.int32)
                    ewf[sl] = jnp.zeros((16,), jnp.float32)
                nck = (cnt + g - 1) // g

                for k in range(ns):
                    @pl.when(nck > k)
                    def _(k=k):
                        fire(k, gidxs[k], rowss[k], semgs[k])

                def gloop(jj, _):
                    j0 = jj * ns
                    for sl in range(ns):
                        @pl.when(j0 + sl < nck)
                        def _(sl=sl, j0=j0):
                            pltpu.make_async_copy(tab.at[gidxs[sl]],
                                                  rowss[sl],
                                                  semgs[sl]).wait()
                            accum(j0 + sl, rowss[sl])

                            @pl.when(j0 + sl + ns < nck)
                            def _():
                                fire(j0 + sl + ns, gidxs[sl], rowss[sl],
                                     semgs[sl])
                    return 0

                lax.fori_loop(0, (nck + ns - 1) // ns, gloop, 0)

                nxt = jnp.minimum(ci + 2, nch - 1)

                @pl.when(h == 0)
                def _():
                    start_stage(nxt, 0, sema)

                @pl.when(h == 1)
                def _():
                    start_stage(nxt, ch, semb)

                return 0

            lax.fori_loop(0, nch, chunk_body, 0)
            wait_stage(0, sema)
            wait_stage(ch, semb)
            pltpu.sync_copy(acc, out.at[pl.ds(base * feat, rpt * feat)])
            return 0

        lax.fori_loop(0, npass, pass_body, 0)

    return seg


_BM = 1000


def _mm_body(a_ref, b_ref, wa_ref, wb_ref, bias_ref, p_ref, h_ref, s_ref):
    h = (jnp.dot(a_ref[...], wa_ref[...], preferred_element_type=jnp.float32)
         + jnp.dot(b_ref[...], wb_ref[...], preferred_element_type=jnp.float32)
         + bias_ref[...])
    h = jnp.maximum(h, 0.0)
    h_ref[...] = h
    p = p_ref[...]
    nrm = jnp.sqrt(jnp.sum(p * p))
    s_ref[...] = jnp.tanh(
        jnp.dot(h, p, preferred_element_type=jnp.float32) / nrm)


def _mm_score(a, b, wa, wb, bias, p):
    kd = a.shape[1]
    grid = N // _BM
    return pl.pallas_call(
        _mm_body,
        grid=(grid,),
        in_specs=[
            pl.BlockSpec((_BM, kd), lambda i: (i, 0)),
            pl.BlockSpec((_BM, kd), lambda i: (i, 0)),
            pl.BlockSpec((kd, HP), lambda i: (0, 0)),
            pl.BlockSpec((kd, HP), lambda i: (0, 0)),
            pl.BlockSpec((1, HP), lambda i: (0, 0)),
            pl.BlockSpec((HP, 1), lambda i: (0, 0)),
        ],
        out_specs=[
            pl.BlockSpec((_BM, HP), lambda i: (i, 0)),
            pl.BlockSpec((_BM, 1), lambda i: (i, 0)),
        ],
        out_shape=[
            jax.ShapeDtypeStruct((N, HP), jnp.float32),
            jax.ShapeDtypeStruct((N, 1), jnp.float32),
        ],
    )(a, b, wa, wb, bias, p)


_TR, _TCL = 8, 1250


def _ikey(v):
    b = lax.bitcast_convert_type(v, jnp.int32)
    return jnp.where(b >= 0, b, b ^ _F32_FLIP)


def _bisect_kth(key, k):
    def step(i, t):
        cand = t + lax.shift_left(jnp.int32(1), jnp.int32(31) - i)
        cnt = jnp.sum(jnp.where(key >= cand, 1, 0))
        return jnp.where(cnt >= k, cand, t)
    return lax.fori_loop(0, 32, step, jnp.int32(_I32_MIN))


def _topk_body(k, s_ref, t_ref, m_ref, sm_ref):
    s = s_ref[...]
    tk = t_ref[...]
    key = _ikey(s)
    kk = jnp.int32(k)

    t0 = _bisect_kth(key, kk)
    gt = key > t0
    eq = key == t0
    extra = kk - jnp.sum(jnp.where(gt, 1, 0))

    key1 = jnp.where(eq, _ikey(tk), jnp.int32(_I32_MIN))
    t1 = _bisect_kth(key1, extra)
    gt1 = eq & (key1 > t1)
    eq1 = eq & (key1 == t1)
    extra1 = extra - jnp.sum(jnp.where(gt1, 1, 0))

    idx = (lax.broadcasted_iota(jnp.int32, (_TR, _TCL), 0) * _TCL
           + lax.broadcasted_iota(jnp.int32, (_TR, _TCL), 1))

    def jstep(_, lohi):
        lo, hi = lohi
        mid = (lo + hi) // 2
        cnt = jnp.sum(jnp.where(eq1 & (idx < mid), 1, 0))
        return (jnp.where(cnt >= extra1, lo, mid),
                jnp.where(cnt >= extra1, mid, hi))

    _, j = lax.fori_loop(0, 15, jstep, (jnp.int32(0), jnp.int32(N)))

    m = gt | gt1 | (eq1 & (idx < j))
    mf = m.astype(jnp.float32)
    m_ref[...] = mf
    sm_ref[...] = s * mf


def _topk(s, tiekey, k):
    return pl.pallas_call(
        functools.partial(_topk_body, k),
        out_shape=[
            jax.ShapeDtypeStruct((_TR, _TCL), jnp.float32),
            jax.ShapeDtypeStruct((_TR, _TCL), jnp.float32),
        ],
    )(s, tiekey)


def _readout_body(store_hm, h_ref, sm_ref, m_ref, *out_refs):
    if store_hm:
        hm_ref, gmax_ref, gsum_ref = out_refs
    else:
        gmax_ref, gsum_ref = out_refs
    i = pl.program_id(0)
    hm = h_ref[...] * sm_ref[...]
    if store_hm:
        hm_ref[...] = hm
    blkmax = jnp.max(jnp.where(m_ref[...] > 0, hm, -3e38), axis=0,
                     keepdims=True)
    blksum = jnp.sum(hm, axis=0, keepdims=True)

    @pl.when(i == 0)
    def _():
        gmax_ref[...] = blkmax
        gsum_ref[...] = blksum

    @pl.when(i > 0)
    def _():
        gmax_ref[...] = jnp.maximum(gmax_ref[...], blkmax)
        gsum_ref[...] = gsum_ref[...] + blksum


def _readout(h, sm, m, store_hm):
    grid = N // _BM
    out_specs = [pl.BlockSpec((1, HP), lambda i: (0, 0)),
                 pl.BlockSpec((1, HP), lambda i: (0, 0))]
    out_shape = [jax.ShapeDtypeStruct((1, HP), jnp.float32),
                 jax.ShapeDtypeStruct((1, HP), jnp.float32)]
    if store_hm:
        out_specs.insert(0, pl.BlockSpec((_BM, HP), lambda i: (i, 0)))
        out_shape.insert(0, jax.ShapeDtypeStruct((N, HP), jnp.float32))
    return pl.pallas_call(
        functools.partial(_readout_body, store_hm),
        grid=(grid,),
        in_specs=[
            pl.BlockSpec((_BM, HP), lambda i: (i, 0)),
            pl.BlockSpec((_BM, 1), lambda i: (i, 0)),
            pl.BlockSpec((_BM, 1), lambda i: (i, 0)),
        ],
        out_specs=out_specs,
        out_shape=out_shape,
    )(h, sm, m)


_L2P = 4096
_BL2 = 512


def _head_body(g1max_ref, g1sum_ref, g2max_ref, g2sum_ref,
               wl1_ref, bl1_ref, wl2_ref, bl2_ref, wl3_ref, bl3_ref,
               o_ref, g_scr, a1_scr):
    kstep = pl.program_id(0)

    @pl.when(kstep == 0)
    def _():
        gmax = g1max_ref[0:1, 0:H] + g2max_ref[0:1, 0:H]
        gmean = (g1sum_ref[0:1, 0:H] / K1) + (g2sum_ref[0:1, 0:H] / K2)
        g = jnp.concatenate([gmax, gmean], axis=1)
        g_scr[...] = g
        a1_scr[...] = jnp.maximum(
            jnp.dot(g, wl1_ref[...], preferred_element_type=jnp.float32)
            + bl1_ref[...], 0.0)
        o_ref[...] = jnp.zeros_like(o_ref)

    a2 = jnp.maximum(
        jnp.dot(a1_scr[...], wl2_ref[...], preferred_element_type=jnp.float32)
        + bl2_ref[...], 0.0)
    o_ref[...] = o_ref[...] + jnp.dot(a2, wl3_ref[...],
                                      preferred_element_type=jnp.float32)

    @pl.when(kstep == (_L2P // _BL2) - 1)
    def _():
        o_ref[...] = jax.nn.sigmoid(o_ref[...] + bl3_ref[...])


def _head(g1max, g1sum, g2max, g2sum, wl1t, bl1, wl2t, bl2, wl3t, bl3):
    grid = _L2P // _BL2
    return pl.pallas_call(
        _head_body,
        grid=(grid,),
        in_specs=[
            pl.BlockSpec((1, HP), lambda i: (0, 0)),
            pl.BlockSpec((1, HP), lambda i: (0, 0)),
            pl.BlockSpec((1, HP), lambda i: (0, 0)),
            pl.BlockSpec((1, HP), lambda i: (0, 0)),
            pl.BlockSpec((2 * H, 2000), lambda i: (0, 0)),
            pl.BlockSpec((1, 2000), lambda i: (0, 0)),
            pl.BlockSpec((2000, _BL2), lambda i: (0, i)),
            pl.BlockSpec((1, _BL2), lambda i: (0, i)),
            pl.BlockSpec((_BL2, OUT), lambda i: (i, 0)),
            pl.BlockSpec((1, OUT), lambda i: (0, 0)),
        ],
        out_specs=pl.BlockSpec((1, OUT), lambda i: (0, 0)),
        out_shape=jax.ShapeDtypeStruct((1, OUT), jnp.float32),
        scratch_shapes=[
            pltpu.VMEM((1, 2 * H), jnp.float32),
            pltpu.VMEM((1, 2000), jnp.float32),
        ],
    )(g1max, g1sum, g2max, g2sum, wl1t, bl1, wl2t, bl2, wl3t, bl3)


_RPT1, _NP1 = 320, 1
_RPT2, _NP2 = 160, 2


@functools.cache
def _segsum(feat, rpt, npass, ch, g, ns):
    return _make_segsum(feat, rpt, npass, ch, g, ns)


@functools.cache
def _edgemask():
    return _make_edgemask()


def _padw(w, rows, cols):
    return jnp.pad(w, ((0, rows - w.shape[0]), (0, cols - w.shape[1])))


def kernel(x, edge_index, edge_attr, W_rel1, b_rel1, W_root1, p1,
           W_rel2, b_rel2, W_root2, p2, W_l1, b_l1, W_l2, b_l2, W_l3, b_l3):
    src = edge_index[0]
    dst = edge_index[1]
    ew = edge_attr

    wr1t = _padw(W_rel1.T, F_IN, HP)
    wt1t = _padw(W_root1.T, F_IN, HP)
    b1p = _padw(b_rel1[None, :], 1, HP)
    p1p = _padw(p1[:, None], HP, 1)
    wr2t = _padw(W_rel2.T, HP, HP)
    wt2t = _padw(W_root2.T, HP, HP)
    b2p = _padw(b_rel2[None, :], 1, HP)
    p2p = _padw(p2[:, None], HP, 1)

    z1 = jnp.zeros((_RPT1 * F_IN,), jnp.float32)
    z2 = jnp.zeros((_RPT2 * HP,), jnp.float32)

    agg1 = _segsum(F_IN, _RPT1, _NP1, 2000, 16, 2)(x, src, dst, ew, z1)
    agg1 = agg1.reshape(-1, F_IN)[:N]
    h, s1 = _mm_score(agg1, x, wr1t, wt1t, b1p, p1p)

    s1r = s1.reshape(_TR, _TCL)
    m1r, sm1r = _topk(s1r, jnp.zeros((_TR, _TCL), jnp.float32), K1)
    m1 = m1r.reshape(N, 1)
    sm1 = sm1r.reshape(N, 1)

    hm, g1max, g1sum = _readout(h, sm1, m1, True)

    dst2 = _edgemask()(m1r.reshape(N), src, dst)
    agg2 = _segsum(HP, _RPT2, _NP2, 2000, 16, 2)(hm, src, dst2, ew, z2)
    agg2 = agg2.reshape(-1, HP)[:N]
    h2, s2 = _mm_score(agg2, hm, wr2t, wt2t, b2p, p2p)

    s2m = jnp.where(m1 > 0, s2, -2.0)
    tie = jnp.where(m1r > 0, s1r, -2.0)
    m2r, sm2r = _topk(s2m.reshape(_TR, _TCL), tie, K2)
    m2 = m2r.reshape(N, 1)
    sm2 = sm2r.reshape(N, 1)

    g2max, g2sum = _readout(h2, sm2, m2, False)

    wl1t = W_l1.T
    wl2t = _padw(W_l2.T, 2000, _L2P)
    bl2p = _padw(b_l2[None, :], 1, _L2P)
    wl3t = _padw(W_l3.T, _L2P, OUT)
    return _head(g1max, g1sum, g2max, g2sum,
                 wl1t, b_l1[None, :], wl2t, bl2p, wl3t, b_l3[None, :])
